# baseline (speedup 1.0000x reference)
import functools

import jax
import jax.numpy as jnp
from jax import lax
from jax.experimental import pallas as pl
from jax.experimental.pallas import tpu as pltpu
from jax.experimental.pallas import tpu_sc as plsc

N = 10000
D = 128
N_PAD = 10240
NC, NS, L = 2, 16, 16
NW = NC * NS
CHUNK = 128
ROWS_PT = N_PAD // NS
TR = 1280


def _wid(c, s):
    return s * NC + c


def _chunk_range(w, NCHUNK):
    BASE = NCHUNK // NW
    REM = NCHUNK % NW
    base = w * BASE + jnp.minimum(w, REM)
    nloc = jnp.where(w < REM, BASE + 1, BASE)
    return base, nloc, BASE, REM


def _load_idx(hbm1d, buf, base, w, BASE, REM):
    pltpu.sync_copy(hbm1d.at[pl.ds(base * CHUNK, BASE * CHUNK)],
                    buf.at[pl.ds(0, BASE * CHUNK)])
    if REM:
        @pl.when(w < REM)
        def _():
            pltpu.sync_copy(hbm1d.at[pl.ds((base + BASE) * CHUNK, CHUNK)],
                            buf.at[pl.ds(BASE * CHUNK, CHUNK)])


def _make_deg(E):
    NCHUNK = E // CHUNK
    NLOC = NCHUNK // NW + (1 if NCHUNK % NW else 0)
    K = 4

    mesh = plsc.VectorSubcoreMesh(core_axis_name="c", subcore_axis_name="s")

    @functools.partial(
        pl.kernel,
        out_type=jax.ShapeDtypeStruct((NC, N_PAD), jnp.float32),
        mesh=mesh,
        scratch_types=[
            pltpu.VMEM((NLOC * CHUNK,), jnp.int32),
            pltpu.VMEM((CHUNK,), jnp.int32),
            pltpu.VMEM((CHUNK,), jnp.int32),
            pltpu.VMEM((CHUNK,), jnp.int32),
            pltpu.VMEM((CHUNK,), jnp.int32),
            pltpu.VMEM((CHUNK,), jnp.float32),
            pltpu.VMEM_SHARED((N_PAD,), jnp.float32),
            pltpu.SemaphoreType.DMA,
        ],
    )
    def deg_k(dst_hbm, dzero_hbm, out_hbm, dstb, d0, d1, d2, d3,
              onesbuf, acc, ssem):
        dbufs = (d0, d1, d2, d3)
        c = lax.axis_index("c")
        s = lax.axis_index("s")
        w = _wid(c, s)
        base, nloc, BASE, REM = _chunk_range(w, NCHUNK)

        for i in range(CHUNK // L):
            onesbuf[pl.ds(i * L, L)] = jnp.ones((L,), jnp.float32)

        pltpu.sync_copy(dzero_hbm, acc.at[pl.ds(s * ROWS_PT, ROWS_PT)])
        _load_idx(dst_hbm, dstb, base, w, BASE, REM)
        plsc.subcore_barrier()

        ngrp = (nloc + K - 1) // K

        @pl.loop(0, ngrp)
        def _grp(g):
            for k, dk in enumerate(dbufs):
                i = g * K + k

                @pl.when(i < nloc)
                def _(i=i, dk=dk):
                    for t in range(CHUNK // L):
                        dk[pl.ds(t * L, L)] = (
                            dstb[pl.ds(i * CHUNK + t * L, L)])
                    pltpu.async_copy(onesbuf, acc.at[dk], ssem, add=True)

            for k, dk in enumerate(dbufs):
                i = g * K + k

                @pl.when(i < nloc)
                def _(dk=dk):
                    pltpu.make_async_copy(onesbuf, acc.at[dk], ssem).wait()

        plsc.subcore_barrier()
        pltpu.sync_copy(acc.at[pl.ds(s * ROWS_PT, ROWS_PT)],
                        out_hbm.at[c, pl.ds(s * ROWS_PT, ROWS_PT)])

    return deg_k


def _make_scatter(E):
    CH = 32
    NCHUNK = E // CH
    NLOC = NCHUNK // NW + (1 if NCHUNK % NW else 0)
    NB = 8

    mesh = plsc.VectorSubcoreMesh(core_axis_name="c", subcore_axis_name="s")

    @functools.partial(
        pl.kernel,
        out_type=jax.ShapeDtypeStruct((NC, N_PAD, D), jnp.float32),
        mesh=mesh,
        scratch_types=[
            pltpu.VMEM((NLOC * CH,), jnp.int32),
            pltpu.VMEM((7 * CH,), jnp.int32),
            pltpu.VMEM((CH,), jnp.int32),
            pltpu.VMEM((NB, CH, D), jnp.float32),
            pltpu.VMEM_SHARED((N_PAD, D), jnp.float32),
            pltpu.SemaphoreType.DMA,
            pltpu.SemaphoreType.DMA,
        ],
    )
    def scatter_k(g_hbm, pk_hbm, zeros_hbm, out_hbm,
                  pkb, srcc, dstc, rows, acc, gsem, ssem):
        c = lax.axis_index("c")
        s = lax.axis_index("s")
        w = _wid(c, s)
        BASE = NCHUNK // NW
        REM = NCHUNK % NW
        base = w * BASE + jnp.minimum(w, REM)
        nloc = jnp.where(w < REM, BASE + 1, BASE)

        def src_slot(i):
            off = pl.multiple_of((i % 7) * CH, CH)
            return srcc.at[pl.ds(off, CH)]

        def unpack_src(i):
            sl = src_slot(i)
            for t in range(CH // L):
                v = pkb[pl.ds(i * CH + t * L, L)]
                sl[pl.ds(t * L, L)] = jnp.bitwise_and(v, 0xFFFF)

        def unpack_dst(i):
            for t in range(CH // L):
                v = pkb[pl.ds(i * CH + t * L, L)]
                dstc[pl.ds(t * L, L)] = lax.shift_right_logical(v, 16)

        pltpu.sync_copy(zeros_hbm, acc.at[pl.ds(s * ROWS_PT, ROWS_PT)])
        pltpu.sync_copy(pk_hbm.at[pl.ds(base * CH, BASE * CH)],
                        pkb.at[pl.ds(0, BASE * CH)])
        if REM:
            @pl.when(w < REM)
            def _():
                pltpu.sync_copy(pk_hbm.at[pl.ds((base + BASE) * CH, CH)],
                                pkb.at[pl.ds(BASE * CH, CH)])

        unpack_src(0)
        pltpu.async_copy(g_hbm.at[src_slot(0)], rows.at[0], gsem)

        @pl.when(nloc > 1)
        def _():
            unpack_src(1)
            pltpu.async_copy(g_hbm.at[src_slot(1)], rows.at[1], gsem)

        for p in range(2, 7):
            @pl.when(nloc > p)
            def _(p=p):
                unpack_src(p)
                pltpu.async_copy(g_hbm.at[src_slot(p)], rows.at[p], gsem)

        plsc.subcore_barrier()

        @pl.loop(0, nloc)
        def _edges(i):
            j = i % NB
            pltpu.make_async_copy(g_hbm.at[pl.ds(0, CH)],
                                  rows.at[j], gsem).wait()

            @pl.when(i >= 1)
            def _():
                pltpu.make_async_copy(rows.at[j],
                                      acc.at[pl.ds(0, CH)], ssem).wait()

            @pl.when(i + 7 < nloc)
            def _():
                unpack_src(i + 7)
                pltpu.async_copy(g_hbm.at[src_slot(i + 7)],
                                 rows.at[(i + 7) % NB], gsem)

            unpack_dst(i)
            pltpu.async_copy(rows.at[j], acc.at[dstc], ssem, add=True)

        pltpu.make_async_copy(rows.at[0],
                              acc.at[pl.ds(0, CH)], ssem).wait()
        plsc.subcore_barrier()
        pltpu.sync_copy(acc.at[pl.ds(s * ROWS_PT, ROWS_PT)],
                        out_hbm.at[c, pl.ds(s * ROWS_PT, ROWS_PT)])

    return scatter_k


def _dis_of(dp_ref):
    return lax.rsqrt(dp_ref[0] + dp_ref[1] + 1.0)


def _mm1_body(x_ref, w_ref, dp_ref, o_ref):
    dis = _dis_of(dp_ref)
    o_ref[...] = jnp.dot(x_ref[...], w_ref[...],
                         preferred_element_type=jnp.float32) * dis


def _mm2_body(sp_ref, g_ref, dp_ref, b_ref, w_ref, o_ref):
    dis = _dis_of(dp_ref)
    z = jnp.maximum((sp_ref[0] + sp_ref[1] + g_ref[...]) * dis + b_ref[...],
                    0.0)
    o_ref[...] = jnp.dot(z, w_ref[...],
                         preferred_element_type=jnp.float32) * dis


def _fin_body(sp_ref, g_ref, dp_ref, b_ref, o_ref):
    dis = _dis_of(dp_ref)
    o_ref[...] = (sp_ref[0] + sp_ref[1] + g_ref[...]) * dis + b_ref[...]


_GRID = (N_PAD // TR,)
_bs_rows = pl.BlockSpec((TR, D), lambda i: (i, 0))
_bs_part = pl.BlockSpec((NC, TR, D), lambda i: (0, i, 0))
_bs_deg = pl.BlockSpec((NC, TR, 1), lambda i: (0, i, 0))
_bs_w = pl.BlockSpec((D, D), lambda i: (0, 0))
_bs_b = pl.BlockSpec((1, D), lambda i: (0, 0))
_OUT = jax.ShapeDtypeStruct((N_PAD, D), jnp.float32)

_mm1 = pl.pallas_call(_mm1_body, grid=_GRID, out_shape=_OUT,
                      in_specs=[_bs_rows, _bs_w, _bs_deg],
                      out_specs=_bs_rows)
_mm2 = pl.pallas_call(_mm2_body, grid=_GRID, out_shape=_OUT,
                      in_specs=[_bs_part, _bs_rows, _bs_deg, _bs_b, _bs_w],
                      out_specs=_bs_rows)
_fin = pl.pallas_call(_fin_body, grid=_GRID, out_shape=_OUT,
                      in_specs=[_bs_part, _bs_rows, _bs_deg, _bs_b],
                      out_specs=_bs_rows)


def kernel(x, edge_index, W1, b1, W2, b2):
    E = edge_index.shape[1]
    src = edge_index[0].astype(jnp.int32)
    dst = edge_index[1].astype(jnp.int32)
    packed = jnp.bitwise_or(src, lax.shift_left(dst, 16))

    x_pad = jnp.pad(x, ((0, N_PAD - N), (0, 0)))
    zeros = jnp.zeros((ROWS_PT, D), jnp.float32)
    dzero = jnp.zeros((ROWS_PT,), jnp.float32)
    b1r = b1.reshape(1, D)
    b2r = b2.reshape(1, D)

    degp = _make_deg(E)(dst, dzero)
    degp = degp.reshape(NC, N_PAD, 1)

    scat = _make_scatter(E)
    g1 = _mm1(x_pad, W1, degp)
    s1 = scat(g1, packed, zeros)
    g2 = _mm2(s1, g1, degp, b1r, W2)
    s2 = scat(g2, packed, zeros)
    out = _fin(s2, g2, degp, b2r)
    return out[:N]


# speedup vs baseline: 1.0652x; 1.0652x over previous
import functools

import jax
import jax.numpy as jnp
from jax import lax
from jax.experimental import pallas as pl
from jax.experimental.pallas import tpu as pltpu
from jax.experimental.pallas import tpu_sc as plsc

N = 10000
D = 128
N_PAD = 10240
NC, NS, L = 2, 16, 16
NW = NC * NS
CHUNK = 128
ROWS_PT = N_PAD // NS
TR = 1280


def _wid(c, s):
    return s * NC + c


def _chunk_range(w, NCHUNK):
    BASE = NCHUNK // NW
    REM = NCHUNK % NW
    base = w * BASE + jnp.minimum(w, REM)
    nloc = jnp.where(w < REM, BASE + 1, BASE)
    return base, nloc, BASE, REM


def _load_idx(hbm1d, buf, base, w, BASE, REM):
    pltpu.sync_copy(hbm1d.at[pl.ds(base * CHUNK, BASE * CHUNK)],
                    buf.at[pl.ds(0, BASE * CHUNK)])
    if REM:
        @pl.when(w < REM)
        def _():
            pltpu.sync_copy(hbm1d.at[pl.ds((base + BASE) * CHUNK, CHUNK)],
                            buf.at[pl.ds(BASE * CHUNK, CHUNK)])


def _make_deg(E):
    NCHUNK = E // CHUNK
    NLOC = NCHUNK // NW + (1 if NCHUNK % NW else 0)
    K = 4

    mesh = plsc.VectorSubcoreMesh(core_axis_name="c", subcore_axis_name="s")

    @functools.partial(
        pl.kernel,
        out_type=jax.ShapeDtypeStruct((NC, N_PAD), jnp.float32),
        mesh=mesh,
        scratch_types=[
            pltpu.VMEM((NLOC * CHUNK,), jnp.int32),
            pltpu.VMEM((CHUNK,), jnp.int32),
            pltpu.VMEM((CHUNK,), jnp.int32),
            pltpu.VMEM((CHUNK,), jnp.int32),
            pltpu.VMEM((CHUNK,), jnp.int32),
            pltpu.VMEM((CHUNK,), jnp.float32),
            pltpu.VMEM_SHARED((N_PAD,), jnp.float32),
            pltpu.SemaphoreType.DMA,
        ],
    )
    def deg_k(ei_hbm, dzero_hbm, out_hbm, dstb, d0, d1, d2, d3,
              onesbuf, acc, ssem):
        dst_hbm = ei_hbm.at[1]
        dbufs = (d0, d1, d2, d3)
        c = lax.axis_index("c")
        s = lax.axis_index("s")
        w = _wid(c, s)
        base, nloc, BASE, REM = _chunk_range(w, NCHUNK)

        for i in range(CHUNK // L):
            onesbuf[pl.ds(i * L, L)] = jnp.ones((L,), jnp.float32)

        pltpu.sync_copy(dzero_hbm, acc.at[pl.ds(s * ROWS_PT, ROWS_PT)])
        _load_idx(dst_hbm, dstb, base, w, BASE, REM)
        plsc.subcore_barrier()

        ngrp = (nloc + K - 1) // K

        @pl.loop(0, ngrp)
        def _grp(g):
            for k, dk in enumerate(dbufs):
                i = g * K + k

                @pl.when(i < nloc)
                def _(i=i, dk=dk):
                    for t in range(CHUNK // L):
                        dk[pl.ds(t * L, L)] = (
                            dstb[pl.ds(i * CHUNK + t * L, L)])
                    pltpu.async_copy(onesbuf, acc.at[dk], ssem, add=True)

            for k, dk in enumerate(dbufs):
                i = g * K + k

                @pl.when(i < nloc)
                def _(dk=dk):
                    pltpu.make_async_copy(onesbuf, acc.at[dk], ssem).wait()

        plsc.subcore_barrier()
        pltpu.sync_copy(acc.at[pl.ds(s * ROWS_PT, ROWS_PT)],
                        out_hbm.at[c, pl.ds(s * ROWS_PT, ROWS_PT)])

    return deg_k


def _make_scatter(E):
    CH = 64
    NCHUNK = E // CH
    NLOC = NCHUNK // NW + (1 if NCHUNK % NW else 0)
    NB = 4

    mesh = plsc.VectorSubcoreMesh(core_axis_name="c", subcore_axis_name="s")

    @functools.partial(
        pl.kernel,
        out_type=jax.ShapeDtypeStruct((NC, N_PAD, D), jnp.float32),
        mesh=mesh,
        scratch_types=[
            pltpu.VMEM((NLOC * CH,), jnp.int32),
            pltpu.VMEM((3 * CH,), jnp.int32),
            pltpu.VMEM((CH,), jnp.int32),
            pltpu.VMEM((NB, CH, D), jnp.float32),
            pltpu.VMEM_SHARED((N_PAD, D), jnp.float32),
            pltpu.SemaphoreType.DMA,
            pltpu.SemaphoreType.DMA,
        ],
    )
    def scatter_k(g_hbm, pk_hbm, zeros_hbm, out_hbm,
                  pkb, srcc, dstc, rows, acc, gsem, ssem):
        c = lax.axis_index("c")
        s = lax.axis_index("s")
        w = _wid(c, s)
        BASE = NCHUNK // NW
        REM = NCHUNK % NW
        base = w * BASE + jnp.minimum(w, REM)
        nloc = jnp.where(w < REM, BASE + 1, BASE)

        def src_slot(i):
            off = pl.multiple_of((i % 3) * CH, CH)
            return srcc.at[pl.ds(off, CH)]

        def unpack_src(i):
            sl = src_slot(i)
            for t in range(CH // L):
                v = pkb[pl.ds(i * CH + t * L, L)]
                sl[pl.ds(t * L, L)] = jnp.bitwise_and(v, 0xFFFF)

        def unpack_dst(i):
            for t in range(CH // L):
                v = pkb[pl.ds(i * CH + t * L, L)]
                dstc[pl.ds(t * L, L)] = lax.shift_right_logical(v, 16)

        pltpu.sync_copy(zeros_hbm, acc.at[pl.ds(s * ROWS_PT, ROWS_PT)])
        pltpu.sync_copy(pk_hbm.at[pl.ds(base * CH, BASE * CH)],
                        pkb.at[pl.ds(0, BASE * CH)])
        if REM:
            @pl.when(w < REM)
            def _():
                pltpu.sync_copy(pk_hbm.at[pl.ds((base + BASE) * CH, CH)],
                                pkb.at[pl.ds(BASE * CH, CH)])

        unpack_src(0)
        pltpu.async_copy(g_hbm.at[src_slot(0)], rows.at[0], gsem)

        @pl.when(nloc > 1)
        def _():
            unpack_src(1)
            pltpu.async_copy(g_hbm.at[src_slot(1)], rows.at[1], gsem)

        @pl.when(nloc > 2)
        def _():
            unpack_src(2)
            pltpu.async_copy(g_hbm.at[src_slot(2)], rows.at[2], gsem)

        plsc.subcore_barrier()

        @pl.loop(0, nloc)
        def _edges(i):
            j = i % NB
            pltpu.make_async_copy(g_hbm.at[pl.ds(0, CH)],
                                  rows.at[j], gsem).wait()

            @pl.when(i >= 1)
            def _():
                pltpu.make_async_copy(rows.at[j],
                                      acc.at[pl.ds(0, CH)], ssem).wait()

            @pl.when(i + 3 < nloc)
            def _():
                unpack_src(i + 3)
                pltpu.async_copy(g_hbm.at[src_slot(i + 3)],
                                 rows.at[(i + 3) % NB], gsem)

            unpack_dst(i)
            pltpu.async_copy(rows.at[j], acc.at[dstc], ssem, add=True)

        pltpu.make_async_copy(rows.at[0],
                              acc.at[pl.ds(0, CH)], ssem).wait()
        plsc.subcore_barrier()
        pltpu.sync_copy(acc.at[pl.ds(s * ROWS_PT, ROWS_PT)],
                        out_hbm.at[c, pl.ds(s * ROWS_PT, ROWS_PT)])

    return scatter_k


def _dis_of(dp_ref):
    return lax.rsqrt(dp_ref[0] + dp_ref[1] + 1.0)


def _mm1_body(x_ref, w_ref, dp_ref, o_ref):
    dis = _dis_of(dp_ref)
    o_ref[...] = jnp.dot(x_ref[...], w_ref[...],
                         preferred_element_type=jnp.float32) * dis


def _mm2_body(sp_ref, g_ref, dp_ref, b_ref, w_ref, o_ref):
    dis = _dis_of(dp_ref)
    z = jnp.maximum((sp_ref[0] + sp_ref[1] + g_ref[...]) * dis + b_ref[...],
                    0.0)
    o_ref[...] = jnp.dot(z, w_ref[...],
                         preferred_element_type=jnp.float32) * dis


def _fin_body(sp_ref, g_ref, dp_ref, b_ref, o_ref):
    dis = _dis_of(dp_ref)
    o_ref[...] = (sp_ref[0] + sp_ref[1] + g_ref[...]) * dis + b_ref[...]


_GRID = (N_PAD // TR,)
_bs_rows = pl.BlockSpec((TR, D), lambda i: (i, 0))
_bs_part = pl.BlockSpec((NC, TR, D), lambda i: (0, i, 0))
_bs_deg = pl.BlockSpec((NC, TR, 1), lambda i: (0, i, 0))
_bs_w = pl.BlockSpec((D, D), lambda i: (0, 0))
_bs_b = pl.BlockSpec((1, D), lambda i: (0, 0))
_OUT = jax.ShapeDtypeStruct((N_PAD, D), jnp.float32)
_OUT_N = jax.ShapeDtypeStruct((N, D), jnp.float32)

_mm1 = pl.pallas_call(_mm1_body, grid=_GRID, out_shape=_OUT,
                      in_specs=[_bs_rows, _bs_w, _bs_deg],
                      out_specs=_bs_rows)
_mm2 = pl.pallas_call(_mm2_body, grid=_GRID, out_shape=_OUT,
                      in_specs=[_bs_part, _bs_rows, _bs_deg, _bs_b, _bs_w],
                      out_specs=_bs_rows)
_fin = pl.pallas_call(_fin_body, grid=_GRID, out_shape=_OUT_N,
                      in_specs=[_bs_part, _bs_rows, _bs_deg, _bs_b],
                      out_specs=_bs_rows)


def kernel(x, edge_index, W1, b1, W2, b2):
    E = edge_index.shape[1]
    ei = edge_index.astype(jnp.int32)
    packed = jnp.bitwise_or(ei[0], lax.shift_left(ei[1], 16))

    zeros = jnp.zeros((ROWS_PT, D), jnp.float32)
    dzero = jnp.zeros((ROWS_PT,), jnp.float32)
    b1r = b1.reshape(1, D)
    b2r = b2.reshape(1, D)

    degp = _make_deg(E)(ei, dzero)
    degp = degp.reshape(NC, N_PAD, 1)

    scat = _make_scatter(E)
    g1 = _mm1(x, W1, degp)
    s1 = scat(g1, packed, zeros)
    g2 = _mm2(s1, g1, degp, b1r, W2)
    s2 = scat(g2, packed, zeros)
    return _fin(s2, g2, degp, b2r)


# speedup vs baseline: 1.1295x; 1.0603x over previous
import functools

import jax
import jax.numpy as jnp
from jax import lax
from jax.experimental import pallas as pl
from jax.experimental.pallas import tpu as pltpu
from jax.experimental.pallas import tpu_sc as plsc

N = 10000
D = 128
N_PAD = 10240
NC, NS, L = 2, 16, 16
NW = NC * NS
CHUNK = 128
ROWS_PT = N_PAD // NS
TR = 1280


def _wid(c, s):
    return s * NC + c


def _chunk_range(w, NCHUNK):
    BASE = NCHUNK // NW
    REM = NCHUNK % NW
    base = w * BASE + jnp.minimum(w, REM)
    nloc = jnp.where(w < REM, BASE + 1, BASE)
    return base, nloc, BASE, REM


def _load_idx(hbm1d, buf, base, w, BASE, REM):
    pltpu.sync_copy(hbm1d.at[pl.ds(base * CHUNK, BASE * CHUNK)],
                    buf.at[pl.ds(0, BASE * CHUNK)])
    if REM:
        @pl.when(w < REM)
        def _():
            pltpu.sync_copy(hbm1d.at[pl.ds((base + BASE) * CHUNK, CHUNK)],
                            buf.at[pl.ds(BASE * CHUNK, CHUNK)])


def _make_deg(E):
    NCHUNK = E // CHUNK
    NLOC = NCHUNK // NW + (1 if NCHUNK % NW else 0)
    K = 4

    mesh = plsc.VectorSubcoreMesh(core_axis_name="c", subcore_axis_name="s")

    @functools.partial(
        pl.kernel,
        out_type=(jax.ShapeDtypeStruct((NC, N_PAD), jnp.float32),
                  jax.ShapeDtypeStruct((E,), jnp.int32)),
        mesh=mesh,
        scratch_types=[
            pltpu.VMEM((NLOC * CHUNK,), jnp.int32),
            pltpu.VMEM((NLOC * CHUNK,), jnp.int32),
            pltpu.VMEM((CHUNK,), jnp.int32),
            pltpu.VMEM((CHUNK,), jnp.int32),
            pltpu.VMEM((CHUNK,), jnp.int32),
            pltpu.VMEM((CHUNK,), jnp.int32),
            pltpu.VMEM((CHUNK,), jnp.float32),
            pltpu.VMEM_SHARED((N_PAD,), jnp.float32),
            pltpu.SemaphoreType.DMA,
        ],
    )
    def deg_k(ei_hbm, dzero_hbm, out_hbm, pk_out, dstb, srcb, d0, d1, d2, d3,
              onesbuf, acc, ssem):
        dst_hbm = ei_hbm.at[1]
        src_hbm = ei_hbm.at[0]
        dbufs = (d0, d1, d2, d3)
        c = lax.axis_index("c")
        s = lax.axis_index("s")
        w = _wid(c, s)
        base, nloc, BASE, REM = _chunk_range(w, NCHUNK)

        for i in range(CHUNK // L):
            onesbuf[pl.ds(i * L, L)] = jnp.ones((L,), jnp.float32)

        pltpu.sync_copy(dzero_hbm, acc.at[pl.ds(s * ROWS_PT, ROWS_PT)])
        _load_idx(dst_hbm, dstb, base, w, BASE, REM)
        _load_idx(src_hbm, srcb, base, w, BASE, REM)
        plsc.subcore_barrier()

        ngrp = (nloc + K - 1) // K

        @pl.loop(0, ngrp)
        def _grp(g):
            for k, dk in enumerate(dbufs):
                i = g * K + k

                @pl.when(i < nloc)
                def _(i=i, dk=dk):
                    for t in range(CHUNK // L):
                        dk[pl.ds(t * L, L)] = (
                            dstb[pl.ds(i * CHUNK + t * L, L)])
                    pltpu.async_copy(onesbuf, acc.at[dk], ssem, add=True)

            for k, dk in enumerate(dbufs):
                i = g * K + k

                @pl.when(i < nloc)
                def _(dk=dk):
                    pltpu.make_async_copy(onesbuf, acc.at[dk], ssem).wait()

        @pl.loop(0, nloc * (CHUNK // L))
        def _pack(t):
            sl = pl.ds(t * L, L)
            srcb[sl] = jnp.bitwise_or(
                srcb[sl], lax.shift_left(dstb[sl], 16))

        pltpu.sync_copy(srcb.at[pl.ds(0, BASE * CHUNK)],
                        pk_out.at[pl.ds(base * CHUNK, BASE * CHUNK)])
        if REM:
            @pl.when(w < REM)
            def _():
                pltpu.sync_copy(
                    srcb.at[pl.ds(BASE * CHUNK, CHUNK)],
                    pk_out.at[pl.ds((base + BASE) * CHUNK, CHUNK)])

        plsc.subcore_barrier()
        pltpu.sync_copy(acc.at[pl.ds(s * ROWS_PT, ROWS_PT)],
                        out_hbm.at[c, pl.ds(s * ROWS_PT, ROWS_PT)])

    return deg_k


def _make_scatter(E):
    CH = 64
    NCHUNK = E // CH
    NLOC = NCHUNK // NW + (1 if NCHUNK % NW else 0)
    NB = 4

    mesh = plsc.VectorSubcoreMesh(core_axis_name="c", subcore_axis_name="s")

    @functools.partial(
        pl.kernel,
        out_type=jax.ShapeDtypeStruct((NC, N_PAD, D), jnp.float32),
        mesh=mesh,
        scratch_types=[
            pltpu.VMEM((NLOC * CH,), jnp.int32),
            pltpu.VMEM((3 * CH,), jnp.int32),
            pltpu.VMEM((CH,), jnp.int32),
            pltpu.VMEM((NB, CH, D), jnp.float32),
            pltpu.VMEM_SHARED((N_PAD, D), jnp.float32),
            pltpu.SemaphoreType.DMA,
            pltpu.SemaphoreType.DMA,
        ],
    )
    def scatter_k(g_hbm, pk_hbm, zeros_hbm, out_hbm,
                  pkb, srcc, dstc, rows, acc, gsem, ssem):
        c = lax.axis_index("c")
        s = lax.axis_index("s")
        w = _wid(c, s)
        BASE = NCHUNK // NW
        REM = NCHUNK % NW
        base = w * BASE + jnp.minimum(w, REM)
        nloc = jnp.where(w < REM, BASE + 1, BASE)

        def src_slot(i):
            off = pl.multiple_of((i % 3) * CH, CH)
            return srcc.at[pl.ds(off, CH)]

        def unpack_src(i):
            sl = src_slot(i)
            for t in range(CH // L):
                v = pkb[pl.ds(i * CH + t * L, L)]
                sl[pl.ds(t * L, L)] = jnp.bitwise_and(v, 0xFFFF)

        def unpack_dst(i):
            for t in range(CH // L):
                v = pkb[pl.ds(i * CH + t * L, L)]
                dstc[pl.ds(t * L, L)] = lax.shift_right_logical(v, 16)

        pltpu.sync_copy(zeros_hbm, acc.at[pl.ds(s * ROWS_PT, ROWS_PT)])
        pltpu.sync_copy(pk_hbm.at[pl.ds(base * CH, BASE * CH)],
                        pkb.at[pl.ds(0, BASE * CH)])
        if REM:
            @pl.when(w < REM)
            def _():
                pltpu.sync_copy(pk_hbm.at[pl.ds((base + BASE) * CH, CH)],
                                pkb.at[pl.ds(BASE * CH, CH)])

        unpack_src(0)
        pltpu.async_copy(g_hbm.at[src_slot(0)], rows.at[0], gsem)

        @pl.when(nloc > 1)
        def _():
            unpack_src(1)
            pltpu.async_copy(g_hbm.at[src_slot(1)], rows.at[1], gsem)

        @pl.when(nloc > 2)
        def _():
            unpack_src(2)
            pltpu.async_copy(g_hbm.at[src_slot(2)], rows.at[2], gsem)

        plsc.subcore_barrier()

        @pl.loop(0, nloc)
        def _edges(i):
            j = i % NB
            pltpu.make_async_copy(g_hbm.at[pl.ds(0, CH)],
                                  rows.at[j], gsem).wait()

            @pl.when(i >= 1)
            def _():
                pltpu.make_async_copy(rows.at[j],
                                      acc.at[pl.ds(0, CH)], ssem).wait()

            @pl.when(i + 3 < nloc)
            def _():
                unpack_src(i + 3)
                pltpu.async_copy(g_hbm.at[src_slot(i + 3)],
                                 rows.at[(i + 3) % NB], gsem)

            unpack_dst(i)
            pltpu.async_copy(rows.at[j], acc.at[dstc], ssem, add=True)

        pltpu.make_async_copy(rows.at[0],
                              acc.at[pl.ds(0, CH)], ssem).wait()
        plsc.subcore_barrier()
        pltpu.sync_copy(acc.at[pl.ds(s * ROWS_PT, ROWS_PT)],
                        out_hbm.at[c, pl.ds(s * ROWS_PT, ROWS_PT)])

    return scatter_k


def _dis_of(dp_ref):
    d = dp_ref[0] + dp_ref[1] + 1.0
    dis = lax.rsqrt(d)
    dis3 = lax.broadcast_in_dim(dis, (N_PAD // 128, 128, D), (0, 1))
    return dis3.reshape(N_PAD, D)


def _mm1_body(x_ref, w_ref, dp_ref, o_ref):
    dis = _dis_of(dp_ref)
    o_ref[...] = jnp.dot(x_ref[...], w_ref[...],
                         preferred_element_type=jnp.float32) * dis


def _mm2_body(sp_ref, g_ref, dp_ref, b_ref, w_ref, o_ref):
    dis = _dis_of(dp_ref)
    z = jnp.maximum((sp_ref[0] + sp_ref[1] + g_ref[...]) * dis + b_ref[...],
                    0.0)
    o_ref[...] = jnp.dot(z, w_ref[...],
                         preferred_element_type=jnp.float32) * dis


def _fin_body(sp_ref, g_ref, dp_ref, b_ref, o_ref):
    dis = _dis_of(dp_ref)
    o_ref[...] = (sp_ref[0] + sp_ref[1] + g_ref[...]) * dis + b_ref[...]


_GRID = (1,)
_bs_rows = pl.BlockSpec((N_PAD, D), lambda i: (0, 0))
_bs_part = pl.BlockSpec((NC, N_PAD, D), lambda i: (0, 0, 0))
_bs_deg = pl.BlockSpec((NC, N_PAD // 128, 128), lambda i: (0, 0, 0))
_bs_w = pl.BlockSpec((D, D), lambda i: (0, 0))
_bs_b = pl.BlockSpec((1, D), lambda i: (0, 0))
_OUT = jax.ShapeDtypeStruct((N_PAD, D), jnp.float32)
_OUT_N = jax.ShapeDtypeStruct((N, D), jnp.float32)

_mm1 = pl.pallas_call(_mm1_body, grid=_GRID, out_shape=_OUT,
                      in_specs=[_bs_rows, _bs_w, _bs_deg],
                      out_specs=_bs_rows)
_mm2 = pl.pallas_call(_mm2_body, grid=_GRID, out_shape=_OUT,
                      in_specs=[_bs_part, _bs_rows, _bs_deg, _bs_b, _bs_w],
                      out_specs=_bs_rows)
_fin = pl.pallas_call(_fin_body, grid=_GRID, out_shape=_OUT_N,
                      in_specs=[_bs_part, _bs_rows, _bs_deg, _bs_b],
                      out_specs=_bs_rows)


def kernel(x, edge_index, W1, b1, W2, b2):
    E = edge_index.shape[1]
    ei = edge_index.astype(jnp.int32)

    zeros = jnp.zeros((ROWS_PT, D), jnp.float32)
    dzero = jnp.zeros((ROWS_PT,), jnp.float32)
    b1r = b1.reshape(1, D)
    b2r = b2.reshape(1, D)

    degp, packed = _make_deg(E)(ei, dzero)
    degp = degp.reshape(NC, N_PAD // 128, 128)

    scat = _make_scatter(E)
    g1 = _mm1(x, W1, degp)
    s1 = scat(g1, packed, zeros)
    g2 = _mm2(s1, g1, degp, b1r, W2)
    s2 = scat(g2, packed, zeros)
    return _fin(s2, g2, degp, b2r)


# speedup vs baseline: 1.1451x; 1.0138x over previous
import functools

import jax
import jax.numpy as jnp
from jax import lax
from jax.experimental import pallas as pl
from jax.experimental.pallas import tpu as pltpu
from jax.experimental.pallas import tpu_sc as plsc

N = 10000
D = 128
N_PAD = 10240
NC, NS, L = 2, 16, 16
NW = NC * NS
CHUNK = 128
ROWS_PT = N_PAD // NS
TR = 1280


def _wid(c, s):
    return s * NC + c


def _chunk_range(w, NCHUNK):
    BASE = NCHUNK // NW
    REM = NCHUNK % NW
    base = w * BASE + jnp.minimum(w, REM)
    nloc = jnp.where(w < REM, BASE + 1, BASE)
    return base, nloc, BASE, REM


def _load_idx(hbm1d, buf, base, w, BASE, REM):
    pltpu.sync_copy(hbm1d.at[pl.ds(base * CHUNK, BASE * CHUNK)],
                    buf.at[pl.ds(0, BASE * CHUNK)])
    if REM:
        @pl.when(w < REM)
        def _():
            pltpu.sync_copy(hbm1d.at[pl.ds((base + BASE) * CHUNK, CHUNK)],
                            buf.at[pl.ds(BASE * CHUNK, CHUNK)])


def _make_deg(E):
    NCHUNK = E // CHUNK
    NLOC = NCHUNK // NW + (1 if NCHUNK % NW else 0)
    K = 4

    mesh = plsc.VectorSubcoreMesh(core_axis_name="c", subcore_axis_name="s")

    @functools.partial(
        pl.kernel,
        out_type=(jax.ShapeDtypeStruct((NC, N_PAD), jnp.float32),
                  jax.ShapeDtypeStruct((E,), jnp.int32)),
        mesh=mesh,
        scratch_types=[
            pltpu.VMEM((NLOC * CHUNK,), jnp.int32),
            pltpu.VMEM((NLOC * CHUNK,), jnp.int32),
            pltpu.VMEM((CHUNK,), jnp.int32),
            pltpu.VMEM((CHUNK,), jnp.int32),
            pltpu.VMEM((CHUNK,), jnp.int32),
            pltpu.VMEM((CHUNK,), jnp.int32),
            pltpu.VMEM((CHUNK,), jnp.float32),
            pltpu.VMEM_SHARED((N_PAD,), jnp.float32),
            pltpu.SemaphoreType.DMA,
        ],
    )
    def deg_k(ei_hbm, dzero_hbm, out_hbm, pk_out, dstb, srcb, d0, d1, d2, d3,
              onesbuf, acc, ssem):
        dst_hbm = ei_hbm.at[1]
        src_hbm = ei_hbm.at[0]
        dbufs = (d0, d1, d2, d3)
        c = lax.axis_index("c")
        s = lax.axis_index("s")
        w = _wid(c, s)
        base, nloc, BASE, REM = _chunk_range(w, NCHUNK)

        for i in range(CHUNK // L):
            onesbuf[pl.ds(i * L, L)] = jnp.ones((L,), jnp.float32)

        pltpu.sync_copy(dzero_hbm, acc.at[pl.ds(s * ROWS_PT, ROWS_PT)])
        _load_idx(dst_hbm, dstb, base, w, BASE, REM)
        _load_idx(src_hbm, srcb, base, w, BASE, REM)
        plsc.subcore_barrier()

        ngrp = (nloc + K - 1) // K

        @pl.loop(0, ngrp)
        def _grp(g):
            for k, dk in enumerate(dbufs):
                i = g * K + k

                @pl.when(i < nloc)
                def _(i=i, dk=dk):
                    for t in range(CHUNK // L):
                        dk[pl.ds(t * L, L)] = (
                            dstb[pl.ds(i * CHUNK + t * L, L)])
                    pltpu.async_copy(onesbuf, acc.at[dk], ssem, add=True)

            for k, dk in enumerate(dbufs):
                i = g * K + k

                @pl.when(i < nloc)
                def _(dk=dk):
                    pltpu.make_async_copy(onesbuf, acc.at[dk], ssem).wait()

        @pl.loop(0, nloc * (CHUNK // L))
        def _pack(t):
            sl = pl.ds(t * L, L)
            srcb[sl] = jnp.bitwise_or(
                srcb[sl], lax.shift_left(dstb[sl], 16))

        pltpu.sync_copy(srcb.at[pl.ds(0, BASE * CHUNK)],
                        pk_out.at[pl.ds(base * CHUNK, BASE * CHUNK)])
        if REM:
            @pl.when(w < REM)
            def _():
                pltpu.sync_copy(
                    srcb.at[pl.ds(BASE * CHUNK, CHUNK)],
                    pk_out.at[pl.ds((base + BASE) * CHUNK, CHUNK)])

        plsc.subcore_barrier()
        pltpu.sync_copy(acc.at[pl.ds(s * ROWS_PT, ROWS_PT)],
                        out_hbm.at[c, pl.ds(s * ROWS_PT, ROWS_PT)])

    return deg_k


def _make_scatter(E):
    CH = 64
    NCHUNK = E // CH
    NLOC = NCHUNK // NW + (1 if NCHUNK % NW else 0)
    NB = 4

    mesh = plsc.VectorSubcoreMesh(core_axis_name="c", subcore_axis_name="s")

    @functools.partial(
        pl.kernel,
        out_type=jax.ShapeDtypeStruct((NC, N_PAD, D), jnp.float32),
        mesh=mesh,
        scratch_types=[
            pltpu.VMEM((NLOC * CH,), jnp.int32),
            pltpu.VMEM((3 * CH,), jnp.int32),
            pltpu.VMEM((CH,), jnp.int32),
            pltpu.VMEM((NB, CH, D), jnp.float32),
            pltpu.VMEM_SHARED((N_PAD, D), jnp.float32),
            pltpu.SemaphoreType.DMA,
            pltpu.SemaphoreType.DMA,
        ],
    )
    def scatter_k(g_hbm, pk_hbm, zeros_hbm, out_hbm,
                  pkb, srcc, dstc, rows, acc, gsem, ssem):
        c = lax.axis_index("c")
        s = lax.axis_index("s")
        w = _wid(c, s)
        BASE = NCHUNK // NW
        REM = NCHUNK % NW
        base = w * BASE + jnp.minimum(w, REM)
        nloc = jnp.where(w < REM, BASE + 1, BASE)

        def src_slot(i):
            off = pl.multiple_of((i % 3) * CH, CH)
            return srcc.at[pl.ds(off, CH)]

        def unpack_src(i):
            sl = src_slot(i)
            for t in range(CH // L):
                v = pkb[pl.ds(i * CH + t * L, L)]
                sl[pl.ds(t * L, L)] = jnp.bitwise_and(v, 0xFFFF)

        def unpack_dst(i):
            for t in range(CH // L):
                v = pkb[pl.ds(i * CH + t * L, L)]
                dstc[pl.ds(t * L, L)] = lax.shift_right_logical(v, 16)

        zinit = pltpu.async_copy(zeros_hbm,
                                 acc.at[pl.ds(s * ROWS_PT, ROWS_PT)], ssem)
        pltpu.sync_copy(pk_hbm.at[pl.ds(base * CH, BASE * CH)],
                        pkb.at[pl.ds(0, BASE * CH)])
        if REM:
            @pl.when(w < REM)
            def _():
                pltpu.sync_copy(pk_hbm.at[pl.ds((base + BASE) * CH, CH)],
                                pkb.at[pl.ds(BASE * CH, CH)])

        unpack_src(0)
        pltpu.async_copy(g_hbm.at[src_slot(0)], rows.at[0], gsem)

        @pl.when(nloc > 1)
        def _():
            unpack_src(1)
            pltpu.async_copy(g_hbm.at[src_slot(1)], rows.at[1], gsem)

        @pl.when(nloc > 2)
        def _():
            unpack_src(2)
            pltpu.async_copy(g_hbm.at[src_slot(2)], rows.at[2], gsem)

        zinit.wait()
        plsc.subcore_barrier()

        @pl.loop(0, nloc)
        def _edges(i):
            j = i % NB
            pltpu.make_async_copy(g_hbm.at[pl.ds(0, CH)],
                                  rows.at[j], gsem).wait()

            @pl.when(i >= 1)
            def _():
                pltpu.make_async_copy(rows.at[j],
                                      acc.at[pl.ds(0, CH)], ssem).wait()

            @pl.when(i + 3 < nloc)
            def _():
                unpack_src(i + 3)
                pltpu.async_copy(g_hbm.at[src_slot(i + 3)],
                                 rows.at[(i + 3) % NB], gsem)

            unpack_dst(i)
            pltpu.async_copy(rows.at[j], acc.at[dstc], ssem, add=True)

        pltpu.make_async_copy(rows.at[0],
                              acc.at[pl.ds(0, CH)], ssem).wait()
        plsc.subcore_barrier()
        pltpu.sync_copy(acc.at[pl.ds(s * ROWS_PT, ROWS_PT)],
                        out_hbm.at[c, pl.ds(s * ROWS_PT, ROWS_PT)])

    return scatter_k


def _dis_of(dp_ref):
    d = dp_ref[0] + dp_ref[1] + 1.0
    dis = lax.rsqrt(d)
    dis3 = lax.broadcast_in_dim(dis, (N_PAD // 128, 128, D), (0, 1))
    return dis3.reshape(N_PAD, D)


def _mm1_body(x_ref, w_ref, dp_ref, o_ref):
    dis = _dis_of(dp_ref)
    o_ref[...] = jnp.dot(x_ref[...], w_ref[...],
                         preferred_element_type=jnp.float32) * dis


def _mm2_body(sp_ref, g_ref, dp_ref, b_ref, w_ref, o_ref):
    dis = _dis_of(dp_ref)
    z = jnp.maximum((sp_ref[0] + sp_ref[1] + g_ref[...]) * dis + b_ref[...],
                    0.0)
    o_ref[...] = jnp.dot(z, w_ref[...],
                         preferred_element_type=jnp.float32) * dis


def _fin_body(sp_ref, g_ref, dp_ref, b_ref, o_ref):
    dis = _dis_of(dp_ref)
    o_ref[...] = (sp_ref[0] + sp_ref[1] + g_ref[...]) * dis + b_ref[...]


_GRID = (1,)
_bs_rows = pl.BlockSpec((N_PAD, D), lambda i: (0, 0))
_bs_part = pl.BlockSpec((NC, N_PAD, D), lambda i: (0, 0, 0))
_bs_deg = pl.BlockSpec((NC, N_PAD // 128, 128), lambda i: (0, 0, 0))
_bs_w = pl.BlockSpec((D, D), lambda i: (0, 0))
_bs_b = pl.BlockSpec((1, D), lambda i: (0, 0))
_OUT = jax.ShapeDtypeStruct((N_PAD, D), jnp.float32)
_OUT_N = jax.ShapeDtypeStruct((N, D), jnp.float32)

_mm1 = pl.pallas_call(_mm1_body, grid=_GRID, out_shape=_OUT,
                      in_specs=[_bs_rows, _bs_w, _bs_deg],
                      out_specs=_bs_rows)
_mm2 = pl.pallas_call(_mm2_body, grid=_GRID, out_shape=_OUT,
                      in_specs=[_bs_part, _bs_rows, _bs_deg, _bs_b, _bs_w],
                      out_specs=_bs_rows)
_fin = pl.pallas_call(_fin_body, grid=_GRID, out_shape=_OUT_N,
                      in_specs=[_bs_part, _bs_rows, _bs_deg, _bs_b],
                      out_specs=_bs_rows)


def kernel(x, edge_index, W1, b1, W2, b2):
    E = edge_index.shape[1]
    ei = edge_index.astype(jnp.int32)

    zeros = jnp.zeros((ROWS_PT, D), jnp.float32)
    dzero = jnp.zeros((ROWS_PT,), jnp.float32)
    b1r = b1.reshape(1, D)
    b2r = b2.reshape(1, D)

    degp, packed = _make_deg(E)(ei, dzero)
    degp = degp.reshape(NC, N_PAD // 128, 128)

    scat = _make_scatter(E)
    g1 = _mm1(x, W1, degp)
    s1 = scat(g1, packed, zeros)
    g2 = _mm2(s1, g1, degp, b1r, W2)
    s2 = scat(g2, packed, zeros)
    return _fin(s2, g2, degp, b2r)
